# baseline (speedup 1.0000x reference)
import functools

import jax
import jax.numpy as jnp
import numpy as np
from jax import lax
from jax.experimental import pallas as pl
from jax.experimental.pallas import tpu as pltpu
from jax.experimental.pallas import tpu_sc as plsc

_CLS_NUM_LIST = [5000 // (i + 1) for i in range(100)]
_MAX_M = 0.5
_S = 30.0

_B = 16384
_C = 100
_NW = 32
_RPW = _B // _NW
_NG = _RPW // 16
_JU = 20

_LN2 = 0.6931471805599453


def _poly_log(s):
    bits = plsc.bitcast(s, jnp.int32)
    e = ((bits >> 23) & 255) - 127
    mant = plsc.bitcast((bits & 0x7FFFFF) | 0x3F800000, jnp.float32)
    t = (mant - 1.0) / (mant + 1.0)
    t2 = t * t
    p = jnp.float32(1.0 / 9.0)
    for c in (1.0 / 7.0, 1.0 / 5.0, 1.0 / 3.0, 1.0):
        p = p * t2 + jnp.float32(c)
    return e.astype(jnp.float32) * jnp.float32(_LN2) + (2.0 * t) * p


def _make_sc_kernel():
    mesh = plsc.VectorSubcoreMesh(core_axis_name="c", subcore_axis_name="s")

    @functools.partial(
        pl.kernel,
        mesh=mesh,
        compiler_params=pltpu.CompilerParams(
            needs_layout_passes=False,
            disable_bounds_checks=True,
        ),
        out_type=jax.ShapeDtypeStruct((_NW, 16), jnp.float32),
        scratch_types=[
            pltpu.VMEM((_RPW, _C), jnp.float32),
            pltpu.VMEM((_RPW,), jnp.int32),
            pltpu.VMEM((_RPW,), jnp.float32),
            pltpu.VMEM((128,), jnp.float32),
            pltpu.VMEM((16,), jnp.float32),
        ],
    )
    def k(x_hbm, t_hbm, w_hbm, m_hbm, out_hbm, x_v, t_v, w_v, m_v, acc_v):
        wid = lax.axis_index("s") * 2 + lax.axis_index("c")
        row0 = wid * _RPW
        pltpu.sync_copy(x_hbm.at[pl.ds(row0, _RPW), :], x_v)
        pltpu.sync_copy(t_hbm.at[pl.ds(row0, _RPW)], t_v)
        pltpu.sync_copy(w_hbm.at[pl.ds(row0, _RPW)], w_v)
        pltpu.sync_copy(m_hbm, m_v)

        lane = lax.iota(jnp.int32, 16)
        ninf = jnp.full((16,), -3.0e38, jnp.float32)
        zero = jnp.zeros((16,), jnp.float32)
        one = jnp.full((16,), 1, jnp.int32)

        def group(g, acc):
            rows = g * 16 + lane
            tvec = plsc.load_gather(t_v, [rows])
            wvec = plsc.load_gather(w_v, [rows])
            mt = plsc.load_gather(m_v, [tvec])
            xt = plsc.load_gather(x_v, [rows, tvec])
            xt_m = xt - mt
            plsc.store_scatter(x_v, [rows, tvec], xt_m)

            def p1(jc, carry):
                a, b, cols = carry
                for _ in range(_JU // 2):
                    v0 = plsc.load_gather(x_v, [rows, cols])
                    v1 = plsc.load_gather(x_v, [rows, cols + one])
                    a = jnp.maximum(a, v0)
                    b = jnp.maximum(b, v1)
                    cols = cols + 2
                return a, b, cols

            a, b, _ = lax.fori_loop(
                0, _C // _JU, p1, (ninf, ninf, jnp.zeros((16,), jnp.int32))
            )
            mx = jnp.maximum(a, b)
            big_m = jnp.float32(_S) * mx

            def p2(jc, carry):
                s0, s1, cols = carry
                for _ in range(_JU // 2):
                    v0 = plsc.load_gather(x_v, [rows, cols])
                    v1 = plsc.load_gather(x_v, [rows, cols + one])
                    s0 = s0 + jnp.exp(jnp.float32(_S) * v0 - big_m)
                    s1 = s1 + jnp.exp(jnp.float32(_S) * v1 - big_m)
                    cols = cols + 2
                return s0, s1, cols

            s0, s1, _ = lax.fori_loop(
                0, _C // _JU, p2, (zero, zero, jnp.zeros((16,), jnp.int32))
            )
            s = s0 + s1

            ce = _poly_log(s) + big_m - jnp.float32(_S) * xt_m
            return acc + ce * wvec

        acc_v[...] = lax.fori_loop(0, _NG, group, zero)
        pltpu.sync_copy(acc_v, out_hbm.at[wid])

    return k


def kernel(x, target, instance_weights):
    assert x.shape == (_B, _C) and x.dtype == jnp.float32
    m_np = 1.0 / np.sqrt(np.sqrt(np.array(_CLS_NUM_LIST, dtype=np.float64)))
    m_np = m_np * (_MAX_M / np.max(m_np))
    m_pad = np.zeros(128, np.float32)
    m_pad[:_C] = m_np
    m_list = jnp.asarray(m_pad)

    partials = _make_sc_kernel()(
        x,
        target.astype(jnp.int32),
        instance_weights,
        m_list,
    )
    return jnp.sum(partials) * jnp.float32(1.0 / _B)


# speedup vs baseline: 1.6461x; 1.6461x over previous
import functools

import jax
import jax.numpy as jnp
import numpy as np
from jax import lax
from jax.experimental import pallas as pl
from jax.experimental.pallas import tpu as pltpu
from jax.experimental.pallas import tpu_sc as plsc

_CLS_NUM_LIST = [5000 // (i + 1) for i in range(100)]
_MAX_M = 0.5
_S = 30.0

_B = 16384
_C = 100
_NW = 32
_CPW = _B // _NW
_NCHUNK = 4
_CCOLS = _CPW // _NCHUNK
_GPC = _CCOLS // 16

_LN2 = 0.6931471805599453


def _poly_log(s):
    bits = plsc.bitcast(s, jnp.int32)
    e = ((bits >> 23) & 255) - 127
    mant = plsc.bitcast((bits & 0x7FFFFF) | 0x3F800000, jnp.float32)
    t = (mant - 1.0) / (mant + 1.0)
    t2 = t * t
    p = jnp.float32(1.0 / 9.0)
    for c in (1.0 / 7.0, 1.0 / 5.0, 1.0 / 3.0, 1.0):
        p = p * t2 + jnp.float32(c)
    return e.astype(jnp.float32) * jnp.float32(_LN2) + (2.0 * t) * p


def _make_sc_kernel():
    mesh = plsc.VectorSubcoreMesh(core_axis_name="c", subcore_axis_name="s")

    @functools.partial(
        pl.kernel,
        mesh=mesh,
        compiler_params=pltpu.CompilerParams(
            needs_layout_passes=False,
            disable_bounds_checks=True,
        ),
        out_type=jax.ShapeDtypeStruct((_NW, 16), jnp.float32),
        scratch_types=[
            pltpu.VMEM((_C, _CPW), jnp.float32),
            pltpu.VMEM((_CPW,), jnp.int32),
            pltpu.VMEM((_CPW,), jnp.float32),
            pltpu.VMEM((128,), jnp.float32),
            pltpu.VMEM((16,), jnp.float32),
            pltpu.SemaphoreType.DMA,
            pltpu.SemaphoreType.DMA,
            pltpu.SemaphoreType.DMA,
            pltpu.SemaphoreType.DMA,
        ],
    )
    def k(x_hbm, t_hbm, w_hbm, m_hbm, out_hbm, x_v, t_v, w_v, m_v, acc_v,
          s0, s1, s2, s3):
        wid = lax.axis_index("s") * 2 + lax.axis_index("c")
        col0 = wid * _CPW
        sems = [s0, s1, s2, s3]
        copies = [
            pltpu.async_copy(
                x_hbm.at[:, pl.ds(col0 + kk * _CCOLS, _CCOLS)],
                x_v.at[:, pl.ds(kk * _CCOLS, _CCOLS)],
                sems[kk],
            )
            for kk in range(_NCHUNK)
        ]
        pltpu.sync_copy(t_hbm.at[pl.ds(col0, _CPW)], t_v)
        pltpu.sync_copy(w_hbm.at[pl.ds(col0, _CPW)], w_v)
        pltpu.sync_copy(m_hbm, m_v)

        lane = lax.iota(jnp.int32, 16)
        ninf = jnp.full((16,), -3.0e38, jnp.float32)
        zero = jnp.zeros((16,), jnp.float32)

        def group(g, acc):
            cb = g * 16
            rows_b = cb + lane
            tvec = plsc.load_gather(t_v, [rows_b])
            wvec = plsc.load_gather(w_v, [rows_b])
            mt = plsc.load_gather(m_v, [tvec])
            xt = plsc.load_gather(x_v, [tvec, rows_b])
            xt_m = xt - mt
            plsc.store_scatter(x_v, [tvec, rows_b], xt_m)

            a, b = ninf, ninf
            for j in range(0, _C, 2):
                a = jnp.maximum(a, x_v[j, pl.ds(cb, 16)])
                b = jnp.maximum(b, x_v[j + 1, pl.ds(cb, 16)])
            big_m = jnp.float32(_S) * jnp.maximum(a, b)

            sa, sb = zero, zero
            for j in range(0, _C, 2):
                sa = sa + jnp.exp(
                    jnp.float32(_S) * x_v[j, pl.ds(cb, 16)] - big_m)
                sb = sb + jnp.exp(
                    jnp.float32(_S) * x_v[j + 1, pl.ds(cb, 16)] - big_m)
            s = sa + sb

            ce = _poly_log(s) + big_m - jnp.float32(_S) * xt_m
            return acc + ce * wvec

        acc = zero
        for kk in range(_NCHUNK):
            copies[kk].wait()
            acc = lax.fori_loop(kk * _GPC, (kk + 1) * _GPC, group, acc)

        acc_v[...] = acc
        pltpu.sync_copy(acc_v, out_hbm.at[wid])

    return k


def kernel(x, target, instance_weights):
    assert x.shape == (_B, _C) and x.dtype == jnp.float32
    m_np = 1.0 / np.sqrt(np.sqrt(np.array(_CLS_NUM_LIST, dtype=np.float64)))
    m_np = m_np * (_MAX_M / np.max(m_np))
    m_pad = np.zeros(128, np.float32)
    m_pad[:_C] = m_np
    m_list = jnp.asarray(m_pad)

    partials = _make_sc_kernel()(
        x.T,
        target.astype(jnp.int32),
        instance_weights,
        m_list,
    )
    return jnp.sum(partials) * jnp.float32(1.0 / _B)


# speedup vs baseline: 1.9218x; 1.1674x over previous
import functools

import jax
import jax.numpy as jnp
from jax import lax
from jax.experimental import pallas as pl
from jax.experimental.pallas import tpu as pltpu
from jax.experimental.pallas import tpu_sc as plsc

_CLS_NUM_LIST = [5000 // (i + 1) for i in range(100)]
_MAX_M = 0.5
_S = 30.0

_B = 16384
_C = 100
_NW = 32
_CPW = _B // _NW
_NCHUNK = 2
_CCOLS = _CPW // _NCHUNK
_GPC = _CCOLS // 16

_LN2 = 0.6931471805599453


_M_SCALE = _MAX_M * float(min(_CLS_NUM_LIST)) ** 0.25


def _poly_log(s):
    bits = plsc.bitcast(s, jnp.int32)
    e = ((bits >> 23) & 255) - 127
    mant = plsc.bitcast((bits & 0x7FFFFF) | 0x3F800000, jnp.float32)
    t = (mant - 1.0) / (mant + 1.0)
    t2 = t * t
    p = jnp.float32(1.0 / 9.0)
    for c in (1.0 / 7.0, 1.0 / 5.0, 1.0 / 3.0, 1.0):
        p = p * t2 + jnp.float32(c)
    return e.astype(jnp.float32) * jnp.float32(_LN2) + (2.0 * t) * p


def _make_sc_kernel():
    mesh = plsc.VectorSubcoreMesh(core_axis_name="c", subcore_axis_name="s")

    @functools.partial(
        pl.kernel,
        mesh=mesh,
        compiler_params=pltpu.CompilerParams(
            needs_layout_passes=False,
            disable_bounds_checks=True,
        ),
        out_type=jax.ShapeDtypeStruct((_NW, 16), jnp.float32),
        scratch_types=[
            pltpu.VMEM((_C, _CPW), jnp.float32),
            pltpu.VMEM((_CPW,), jnp.int32),
            pltpu.VMEM((_CPW,), jnp.float32),
            pltpu.VMEM((128,), jnp.float32),
            pltpu.VMEM((16,), jnp.float32),
            pltpu.SemaphoreType.DMA,
            pltpu.SemaphoreType.DMA,
            pltpu.SemaphoreType.DMA,
            pltpu.SemaphoreType.DMA,
        ],
    )
    def k(x_hbm, t_hbm, w_hbm, out_hbm, x_v, t_v, w_v, m_v, acc_v,
          s0, s1, st, sw):
        wid = lax.axis_index("s") * 2 + lax.axis_index("c")
        col0 = wid * _CPW
        sems = [s0, s1]
        copies = [
            pltpu.async_copy(
                x_hbm.at[:, pl.ds(col0 + kk * _CCOLS, _CCOLS)],
                x_v.at[:, pl.ds(kk * _CCOLS, _CCOLS)],
                sems[kk],
            )
            for kk in range(_NCHUNK)
        ]
        t_copy = pltpu.async_copy(t_hbm.at[pl.ds(col0, _CPW)], t_v, st)
        w_copy = pltpu.async_copy(w_hbm.at[pl.ds(col0, _CPW)], w_v, sw)

        lane = lax.iota(jnp.int32, 16)
        for kk in range(8):
            idx1 = (lane + (kk * 16 + 1)).astype(jnp.float32)
            cls = (jnp.float32(5000.0) / idx1).astype(jnp.int32)
            lncls = _poly_log(cls.astype(jnp.float32))
            m_v[pl.ds(kk * 16, 16)] = jnp.float32(_M_SCALE) * jnp.exp(
                jnp.float32(-0.25) * lncls)

        t_copy.wait()
        w_copy.wait()
        ninf = jnp.full((16,), -3.0e38, jnp.float32)
        zero = jnp.zeros((16,), jnp.float32)

        def group(g, acc):
            cb = g * 16
            rows_b = cb + lane
            tvec = plsc.load_gather(t_v, [rows_b])
            wvec = plsc.load_gather(w_v, [rows_b])
            mt = plsc.load_gather(m_v, [tvec])
            xt = plsc.load_gather(x_v, [tvec, rows_b])
            xt_m = xt - mt
            plsc.store_scatter(x_v, [tvec, rows_b], xt_m)

            mx4 = [ninf, ninf, ninf, ninf]
            for j in range(_C):
                mx4[j % 4] = jnp.maximum(mx4[j % 4], x_v[j, pl.ds(cb, 16)])
            big_m = jnp.float32(_S) * jnp.maximum(
                jnp.maximum(mx4[0], mx4[1]), jnp.maximum(mx4[2], mx4[3]))

            sm4 = [zero, zero, zero, zero]
            for j in range(_C):
                sm4[j % 4] = sm4[j % 4] + jnp.exp(
                    jnp.float32(_S) * x_v[j, pl.ds(cb, 16)] - big_m)
            s = (sm4[0] + sm4[1]) + (sm4[2] + sm4[3])

            ce = _poly_log(s) + big_m - jnp.float32(_S) * xt_m
            return acc + ce * wvec

        acc = zero
        for kk in range(_NCHUNK):
            copies[kk].wait()
            acc = lax.fori_loop(kk * _GPC, (kk + 1) * _GPC, group, acc)

        acc_v[...] = acc
        pltpu.sync_copy(acc_v, out_hbm.at[wid])

    return k


def kernel(x, target, instance_weights):
    assert x.shape == (_B, _C) and x.dtype == jnp.float32
    partials = _make_sc_kernel()(
        x.T,
        target.astype(jnp.int32),
        instance_weights,
    )
    return jnp.sum(partials) * jnp.float32(1.0 / _B)


# speedup vs baseline: 1.9895x; 1.0352x over previous
import functools

import jax
import jax.numpy as jnp
from jax import lax
from jax.experimental import pallas as pl
from jax.experimental.pallas import tpu as pltpu
from jax.experimental.pallas import tpu_sc as plsc

_CLS_NUM_LIST = [5000 // (i + 1) for i in range(100)]
_MAX_M = 0.5
_S = 30.0

_B = 16384
_C = 100
_NW = 32
_CPW = _B // _NW
_NCHUNK = 2
_CCOLS = _CPW // _NCHUNK
_GPC = _CCOLS // 16
_CB = 20

_LN2 = 0.6931471805599453


_M_SCALE = _MAX_M * float(min(_CLS_NUM_LIST)) ** 0.25


def _poly_log(s):
    bits = plsc.bitcast(s, jnp.int32)
    e = ((bits >> 23) & 255) - 127
    mant = plsc.bitcast((bits & 0x7FFFFF) | 0x3F800000, jnp.float32)
    t = (mant - 1.0) / (mant + 1.0)
    t2 = t * t
    p = jnp.float32(1.0 / 9.0)
    for c in (1.0 / 7.0, 1.0 / 5.0, 1.0 / 3.0, 1.0):
        p = p * t2 + jnp.float32(c)
    return e.astype(jnp.float32) * jnp.float32(_LN2) + (2.0 * t) * p


def _make_sc_kernel():
    mesh = plsc.VectorSubcoreMesh(core_axis_name="c", subcore_axis_name="s")

    @functools.partial(
        pl.kernel,
        mesh=mesh,
        compiler_params=pltpu.CompilerParams(
            needs_layout_passes=False,
            disable_bounds_checks=True,
        ),
        out_type=jax.ShapeDtypeStruct((_NW, 16), jnp.float32),
        scratch_types=[
            pltpu.VMEM((_C, _CPW), jnp.float32),
            pltpu.VMEM((_CPW,), jnp.int32),
            pltpu.VMEM((_CPW,), jnp.float32),
            pltpu.VMEM((128,), jnp.float32),
            pltpu.VMEM((16,), jnp.float32),
            pltpu.SemaphoreType.DMA,
            pltpu.SemaphoreType.DMA,
            pltpu.SemaphoreType.DMA,
            pltpu.SemaphoreType.DMA,
        ],
    )
    def k(x_hbm, t_hbm, w_hbm, out_hbm, x_v, t_v, w_v, m_v, acc_v,
          s0, s1, st, sw):
        wid = lax.axis_index("s") * 2 + lax.axis_index("c")
        col0 = wid * _CPW
        sems = [s0, s1]
        copies = [
            pltpu.async_copy(
                x_hbm.at[:, pl.ds(col0 + kk * _CCOLS, _CCOLS)],
                x_v.at[:, pl.ds(kk * _CCOLS, _CCOLS)],
                sems[kk],
            )
            for kk in range(_NCHUNK)
        ]
        t_copy = pltpu.async_copy(t_hbm.at[pl.ds(col0, _CPW)], t_v, st)
        w_copy = pltpu.async_copy(w_hbm.at[pl.ds(col0, _CPW)], w_v, sw)

        lane = lax.iota(jnp.int32, 16)
        for kk in range(8):
            idx1 = (lane + (kk * 16 + 1)).astype(jnp.float32)
            cls = (jnp.float32(5000.0) / idx1).astype(jnp.int32)
            lncls = _poly_log(cls.astype(jnp.float32))
            m_v[pl.ds(kk * 16, 16)] = jnp.float32(_M_SCALE) * jnp.exp(
                jnp.float32(-0.25) * lncls)

        t_copy.wait()
        w_copy.wait()
        ninf = jnp.full((16,), -3.0e38, jnp.float32)
        zero = jnp.zeros((16,), jnp.float32)

        def group(g, acc):
            cb = g * 16
            rows_b = cb + lane
            tvec = plsc.load_gather(t_v, [rows_b])
            wvec = plsc.load_gather(w_v, [rows_b])
            mt = plsc.load_gather(m_v, [tvec])
            xt = plsc.load_gather(x_v, [tvec, rows_b])
            xt_m = xt - mt
            plsc.store_scatter(x_v, [tvec, rows_b], xt_m)

            big_m = ninf
            sm4 = [zero, zero, zero, zero]
            for b in range(0, _C, _CB):
                vs = [x_v[j, pl.ds(cb, 16)] for j in range(b, b + _CB)]
                t = list(vs)
                while len(t) > 1:
                    t = [jnp.maximum(t[i], t[i + 1])
                         for i in range(0, len(t) - 1, 2)] + (
                             [t[-1]] if len(t) % 2 else [])
                m2 = jnp.maximum(big_m, t[0])
                bm2 = jnp.float32(_S) * m2
                scale = jnp.exp(jnp.float32(_S) * big_m - bm2)
                sm4 = [sm * scale for sm in sm4]
                for i, v in enumerate(vs):
                    sm4[i % 4] = sm4[i % 4] + jnp.exp(
                        jnp.float32(_S) * v - bm2)
                big_m = m2
            s = (sm4[0] + sm4[1]) + (sm4[2] + sm4[3])

            ce = _poly_log(s) + jnp.float32(_S) * big_m - jnp.float32(
                _S) * xt_m
            return acc + ce * wvec

        acc = zero
        for kk in range(_NCHUNK):
            copies[kk].wait()
            acc = plsc.parallel_loop(
                kk * _GPC, (kk + 1) * _GPC, unroll=1, carry=acc)(group)

        acc_v[...] = acc
        pltpu.sync_copy(acc_v, out_hbm.at[wid])

    return k


def kernel(x, target, instance_weights):
    assert x.shape == (_B, _C) and x.dtype == jnp.float32
    partials = _make_sc_kernel()(
        x.T,
        target.astype(jnp.int32),
        instance_weights,
    )
    return jnp.sum(partials) * jnp.float32(1.0 / _B)


# speedup vs baseline: 2.2086x; 1.1101x over previous
import functools

import jax
import jax.numpy as jnp
from jax import lax
from jax.experimental import pallas as pl
from jax.experimental.pallas import tpu as pltpu
from jax.experimental.pallas import tpu_sc as plsc

_CLS_NUM_LIST = [5000 // (i + 1) for i in range(100)]
_MAX_M = 0.5
_S = 30.0

_B = 16384
_C = 100
_NW = 32
_CPW = _B // _NW
_NCHUNK = 2
_CCOLS = _CPW // _NCHUNK
_GPC = _CCOLS // 16
_CB = 20

_LN2 = 0.6931471805599453


_M_SCALE = _MAX_M * float(min(_CLS_NUM_LIST)) ** 0.25


def _poly_log(s):
    bits = plsc.bitcast(s, jnp.int32)
    e = ((bits >> 23) & 255) - 127
    mant = plsc.bitcast((bits & 0x7FFFFF) | 0x3F800000, jnp.float32)
    t = (mant - 1.0) / (mant + 1.0)
    t2 = t * t
    p = jnp.float32(1.0 / 9.0)
    for c in (1.0 / 7.0, 1.0 / 5.0, 1.0 / 3.0, 1.0):
        p = p * t2 + jnp.float32(c)
    return e.astype(jnp.float32) * jnp.float32(_LN2) + (2.0 * t) * p


def _make_sc_kernel():
    mesh = plsc.VectorSubcoreMesh(core_axis_name="c", subcore_axis_name="s")

    @functools.partial(
        pl.kernel,
        mesh=mesh,
        compiler_params=pltpu.CompilerParams(
            needs_layout_passes=False,
            disable_bounds_checks=True,
        ),
        out_type=jax.ShapeDtypeStruct((_NW, 16), jnp.float32),
        scratch_types=[
            pltpu.VMEM((_C, _CPW), jnp.float32),
            pltpu.VMEM((_CPW,), jnp.int32),
            pltpu.VMEM((_CPW,), jnp.float32),
            pltpu.VMEM((128,), jnp.float32),
            pltpu.VMEM((16,), jnp.float32),
            pltpu.SemaphoreType.DMA,
            pltpu.SemaphoreType.DMA,
            pltpu.SemaphoreType.DMA,
            pltpu.SemaphoreType.DMA,
        ],
    )
    def k(x_hbm, t_hbm, w_hbm, out_hbm, x_v, t_v, w_v, m_v, acc_v,
          s0, s1, st, sw):
        wid = lax.axis_index("s") * 2 + lax.axis_index("c")
        col0 = wid * _CPW
        sems = [s0, s1]
        copies = [
            pltpu.async_copy(
                x_hbm.at[:, pl.ds(col0 + kk * _CCOLS, _CCOLS)],
                x_v.at[:, pl.ds(kk * _CCOLS, _CCOLS)],
                sems[kk],
            )
            for kk in range(_NCHUNK)
        ]
        t_copy = pltpu.async_copy(t_hbm.at[pl.ds(col0, _CPW)], t_v, st)
        w_copy = pltpu.async_copy(w_hbm.at[pl.ds(col0, _CPW)], w_v, sw)

        lane = lax.iota(jnp.int32, 16)
        for kk in range(8):
            idx1 = (lane + (kk * 16 + 1)).astype(jnp.float32)
            cls = (jnp.float32(5000.0) / idx1).astype(jnp.int32)
            lncls = _poly_log(cls.astype(jnp.float32))
            m_v[pl.ds(kk * 16, 16)] = jnp.float32(_M_SCALE) * jnp.exp(
                jnp.float32(-0.25) * lncls)

        t_copy.wait()
        w_copy.wait()
        ninf = jnp.full((16,), -3.0e38, jnp.float32)
        zero = jnp.zeros((16,), jnp.float32)

        def group(g, acc):
            cb = g * 16
            rows_b = cb + lane
            tvec = plsc.load_gather(t_v, [rows_b])
            wvec = plsc.load_gather(w_v, [rows_b])
            mt = plsc.load_gather(m_v, [tvec])
            xt = plsc.load_gather(x_v, [tvec, rows_b])
            xt_m = xt - mt
            plsc.store_scatter(x_v, [tvec, rows_b], xt_m)

            mx4 = [ninf, ninf, ninf, ninf]
            for j in range(_C):
                mx4[j % 4] = jnp.maximum(mx4[j % 4], x_v[j, pl.ds(cb, 16)])
            big_m = jnp.float32(_S) * jnp.maximum(
                jnp.maximum(mx4[0], mx4[1]), jnp.maximum(mx4[2], mx4[3]))

            zero_bf = jnp.zeros((32,), jnp.bfloat16)
            sm4 = [zero_bf, zero_bf, zero_bf, zero_bf]
            for j in range(0, _C, 2):
                a0 = jnp.float32(_S) * x_v[j, pl.ds(cb, 16)] - big_m
                a1 = jnp.float32(_S) * x_v[j + 1, pl.ds(cb, 16)] - big_m
                e = jnp.exp(plsc.pack(a0, a1, format=plsc.PackFormat.INTERLEAVED))
                sm4[(j // 2) % 4] = sm4[(j // 2) % 4] + e
            sbf = (sm4[0] + sm4[1]) + (sm4[2] + sm4[3])
            s0, s1 = plsc.unpack(sbf, format=plsc.PackFormat.INTERLEAVED)
            s = s0 + s1

            ce = _poly_log(s) + big_m - jnp.float32(_S) * xt_m
            return acc + ce * wvec

        acc = zero
        for kk in range(_NCHUNK):
            copies[kk].wait()
            acc = plsc.parallel_loop(
                kk * _GPC, (kk + 1) * _GPC, unroll=2, carry=acc)(group)

        acc_v[...] = acc
        pltpu.sync_copy(acc_v, out_hbm.at[wid])

    return k


def kernel(x, target, instance_weights):
    assert x.shape == (_B, _C) and x.dtype == jnp.float32
    partials = _make_sc_kernel()(
        x.T,
        target.astype(jnp.int32),
        instance_weights,
    )
    return jnp.sum(partials) * jnp.float32(1.0 / _B)
